# baseline (speedup 1.0000x reference)
import functools

import jax
import jax.numpy as jnp
from jax.experimental import pallas as pl
from jax.experimental.pallas import tpu as pltpu

_K = 10
_THRESH = 0.1
_SUBH = 32
_CH0 = 20
_CHF = 4


def _fast_chunk(s_ref, abuf, cbuf, T_ref, C_ref, D_ref, cnt_ref, b, slot,
                base, w):
    nsub = w // _SUBH
    mn = None
    for si in range(nsub):
        rows = pl.ds(si * _SUBH, _SUBH)
        shape = (_SUBH, w)
        T = jnp.ones(shape, jnp.float32)
        cnt = jnp.zeros(shape, jnp.int32)
        C0 = jnp.zeros(shape, jnp.float32)
        C1 = jnp.zeros(shape, jnp.float32)
        C2 = jnp.zeros(shape, jnp.float32)
        D = jnp.zeros(shape, jnp.float32)
        for i in range(_CH0):
            idx = _CH0 - 1 - i
            a = abuf[slot, idx, 0, rows, :]
            take = (a > _THRESH) & (cnt < _K)
            w_ = jnp.where(take, T * a, 0.0)
            C0 = C0 + w_ * cbuf[slot, idx, 0, rows, :]
            C1 = C1 + w_ * cbuf[slot, idx, 1, rows, :]
            C2 = C2 + w_ * cbuf[slot, idx, 2, rows, :]
            D = D + w_ * (s_ref[b, base + idx, 2] * s_ref[b, base + idx, 3])
            T = T - w_
            cnt = cnt + take.astype(jnp.int32)
        T_ref[rows, :] = T
        cnt_ref[rows, :] = cnt
        C_ref[0, rows, :] = C0
        C_ref[1, rows, :] = C1
        C_ref[2, rows, :] = C2
        D_ref[rows, :] = D
        mn = cnt if mn is None else jnp.minimum(mn, cnt)
    return jnp.min(mn)


def _slow_pass(s_ref, hbm_refs, bufs, acc_refs, sem, b, slot, *,
               n_start, n_count, descending, take_visible, with_density, w):
    alpha_hbm, color_hbm = hbm_refs
    abuf, cbuf = bufs
    T_ref, C_ref, D_ref, cnt_ref = acc_refs
    num_chunks = n_count // _CHF

    def copies(j):
        off = ((num_chunks - 1 - j) if descending else j) * _CHF
        start = n_start + off
        return (
            pltpu.make_async_copy(alpha_hbm.at[b, pl.ds(start, _CHF)],
                                  abuf.at[slot, pl.ds(0, _CHF)],
                                  sem.at[slot, 0]),
            pltpu.make_async_copy(color_hbm.at[b, pl.ds(start, _CHF)],
                                  cbuf.at[slot, pl.ds(0, _CHF)],
                                  sem.at[slot, 1]),
            start,
        )

    def cond(state):
        j, done = state
        return jnp.logical_and(jnp.logical_not(done), j < num_chunks)

    def body(state):
        j, _ = state
        ca, cc, start = copies(j)
        ca.start()
        cc.start()
        ca.wait()
        cc.wait()

        def sbody(i, _):
            idx = (_CHF - 1 - i) if descending else i
            a = abuf[slot, idx, 0]
            cnt = cnt_ref[...]
            vis = a > _THRESH
            if not take_visible:
                vis = jnp.logical_not(vis)
            take = vis & (cnt < _K)
            T = T_ref[...]
            w_ = jnp.where(take, T * a, 0.0)
            C_ref[0] += w_ * cbuf[slot, idx, 0]
            C_ref[1] += w_ * cbuf[slot, idx, 1]
            C_ref[2] += w_ * cbuf[slot, idx, 2]
            if with_density:
                D_ref[...] += w_ * (s_ref[b, start + idx, 2] * s_ref[b, start + idx, 3])
            T_ref[...] = T - w_
            cnt_ref[...] = cnt + take.astype(jnp.int32)
            return 0

        jax.lax.fori_loop(0, _CHF, sbody, 0)
        done = jnp.min(cnt_ref[...]) >= _K
        return j + 1, done

    jax.lax.while_loop(cond, body, (jnp.int32(0), jnp.bool_(False)))


def _composite_kernel(s_ref, alpha_hbm, color_hbm, canvas_ref, den_ref,
                      abuf, cbuf, T_ref, C_ref, D_ref, cnt_ref, sem,
                      *, nb, n, w):
    fast_base = n - _CH0

    def start_fast(b):
        sl = b % 3
        pltpu.make_async_copy(alpha_hbm.at[b, pl.ds(fast_base, _CH0)],
                              abuf.at[sl], sem.at[sl, 0]).start()
        pltpu.make_async_copy(color_hbm.at[b, pl.ds(fast_base, _CH0)],
                              cbuf.at[sl], sem.at[sl, 1]).start()

    def wait_fast(b):
        sl = b % 3
        pltpu.make_async_copy(alpha_hbm.at[b, pl.ds(fast_base, _CH0)],
                              abuf.at[sl], sem.at[sl, 0]).wait()
        pltpu.make_async_copy(color_hbm.at[b, pl.ds(fast_base, _CH0)],
                              cbuf.at[sl], sem.at[sl, 1]).wait()

    start_fast(0)
    if nb > 1:
        start_fast(1)

    for b in range(nb):
        if b + 2 < nb:
            start_fast(b + 2)
        wait_fast(b)

        mincnt = _fast_chunk(s_ref, abuf, cbuf, T_ref, C_ref, D_ref,
                             cnt_ref, b, b % 3, fast_base, w)

        hbm_refs = (alpha_hbm, color_hbm)
        bufs = (abuf, cbuf)
        acc_refs = (T_ref, C_ref, D_ref, cnt_ref)

        @pl.when(mincnt < _K)
        def _continue_descending():
            _slow_pass(s_ref, hbm_refs, bufs, acc_refs, sem, b, b % 3,
                       n_start=0, n_count=fast_base, descending=True,
                       take_visible=True, with_density=True, w=w)

            @pl.when(jnp.min(cnt_ref[...]) < _K)
            def _tie_fill():
                _slow_pass(s_ref, hbm_refs, bufs, acc_refs, sem, b, b % 3,
                           n_start=0, n_count=n, descending=False,
                           take_visible=False, with_density=False, w=w)

        T = T_ref[...]
        canvas_ref[b, 0] = C_ref[0] + T
        canvas_ref[b, 1] = C_ref[1] + T
        canvas_ref[b, 2] = C_ref[2] + T
        den_ref[b, 0] = D_ref[...] + T


@jax.jit
def kernel(color_stroke, alpha, params):
    nb, n = color_stroke.shape[0], color_stroke.shape[1]
    w = color_stroke.shape[-1]

    kfn = functools.partial(_composite_kernel, nb=nb, n=n, w=w)

    canvas, den = pl.pallas_call(
        kfn,
        grid=(1,),
        in_specs=[
            pl.BlockSpec((nb, n, 8), lambda i: (0, 0, 0)),
            pl.BlockSpec(memory_space=pl.ANY),
            pl.BlockSpec(memory_space=pl.ANY),
        ],
        out_specs=[
            pl.BlockSpec((nb, 3, w, w), lambda i: (0, 0, 0, 0)),
            pl.BlockSpec((nb, 1, w, w), lambda i: (0, 0, 0, 0)),
        ],
        out_shape=[
            jax.ShapeDtypeStruct((nb, 3, w, w), jnp.float32),
            jax.ShapeDtypeStruct((nb, 1, w, w), jnp.float32),
        ],
        scratch_shapes=[
            pltpu.VMEM((3, _CH0, 1, w, w), jnp.float32),
            pltpu.VMEM((3, _CH0, 3, w, w), jnp.float32),
            pltpu.VMEM((w, w), jnp.float32),
            pltpu.VMEM((3, w, w), jnp.float32),
            pltpu.VMEM((w, w), jnp.float32),
            pltpu.VMEM((w, w), jnp.int32),
            pltpu.SemaphoreType.DMA((3, 2)),
        ],
        compiler_params=pltpu.CompilerParams(
            dimension_semantics=("arbitrary",),
        ),
    )(params, alpha, color_stroke)

    return (canvas, den)


# speedup vs baseline: 1.0040x; 1.0040x over previous
import functools

import jax
import jax.numpy as jnp
from jax.experimental import pallas as pl
from jax.experimental.pallas import tpu as pltpu

_K = 10
_THRESH = 0.1
_SUBH = 64
_CH0 = 20
_CHF = 4


def _fast_chunk(s_ref, abuf, cbuf, T_ref, C_ref, D_ref, cnt_ref, b, slot,
                base, w):
    nsub = w // _SUBH
    mn = None
    for si in range(nsub):
        rows = pl.ds(si * _SUBH, _SUBH)
        shape = (_SUBH, w)
        T = jnp.ones(shape, jnp.float32)
        cnt = jnp.zeros(shape, jnp.int32)
        C0 = jnp.zeros(shape, jnp.float32)
        C1 = jnp.zeros(shape, jnp.float32)
        C2 = jnp.zeros(shape, jnp.float32)
        D = jnp.zeros(shape, jnp.float32)
        for i in range(_CH0):
            idx = _CH0 - 1 - i
            a = abuf[slot, idx, 0, rows, :]
            take = (a > _THRESH) & (cnt < _K)
            w_ = jnp.where(take, T * a, 0.0)
            C0 = C0 + w_ * cbuf[slot, idx, 0, rows, :]
            C1 = C1 + w_ * cbuf[slot, idx, 1, rows, :]
            C2 = C2 + w_ * cbuf[slot, idx, 2, rows, :]
            D = D + w_ * (s_ref[b, base + idx, 2] * s_ref[b, base + idx, 3])
            T = T - w_
            cnt = cnt + take.astype(jnp.int32)
        T_ref[rows, :] = T
        cnt_ref[rows, :] = cnt
        C_ref[0, rows, :] = C0
        C_ref[1, rows, :] = C1
        C_ref[2, rows, :] = C2
        D_ref[rows, :] = D
        mn = cnt if mn is None else jnp.minimum(mn, cnt)
    return jnp.min(mn)


def _slow_pass(s_ref, hbm_refs, bufs, acc_refs, sem, b, slot, *,
               n_start, n_count, descending, take_visible, with_density, w):
    alpha_hbm, color_hbm = hbm_refs
    abuf, cbuf = bufs
    T_ref, C_ref, D_ref, cnt_ref = acc_refs
    num_chunks = n_count // _CHF

    def copies(j):
        off = ((num_chunks - 1 - j) if descending else j) * _CHF
        start = n_start + off
        return (
            pltpu.make_async_copy(alpha_hbm.at[b, pl.ds(start, _CHF)],
                                  abuf.at[slot, pl.ds(0, _CHF)],
                                  sem.at[slot, 0]),
            pltpu.make_async_copy(color_hbm.at[b, pl.ds(start, _CHF)],
                                  cbuf.at[slot, pl.ds(0, _CHF)],
                                  sem.at[slot, 1]),
            start,
        )

    def cond(state):
        j, done = state
        return jnp.logical_and(jnp.logical_not(done), j < num_chunks)

    def body(state):
        j, _ = state
        ca, cc, start = copies(j)
        ca.start()
        cc.start()
        ca.wait()
        cc.wait()

        def sbody(i, _):
            idx = (_CHF - 1 - i) if descending else i
            a = abuf[slot, idx, 0]
            cnt = cnt_ref[...]
            vis = a > _THRESH
            if not take_visible:
                vis = jnp.logical_not(vis)
            take = vis & (cnt < _K)
            T = T_ref[...]
            w_ = jnp.where(take, T * a, 0.0)
            C_ref[0] += w_ * cbuf[slot, idx, 0]
            C_ref[1] += w_ * cbuf[slot, idx, 1]
            C_ref[2] += w_ * cbuf[slot, idx, 2]
            if with_density:
                D_ref[...] += w_ * (s_ref[b, start + idx, 2] * s_ref[b, start + idx, 3])
            T_ref[...] = T - w_
            cnt_ref[...] = cnt + take.astype(jnp.int32)
            return 0

        jax.lax.fori_loop(0, _CHF, sbody, 0)
        done = jnp.min(cnt_ref[...]) >= _K
        return j + 1, done

    jax.lax.while_loop(cond, body, (jnp.int32(0), jnp.bool_(False)))


def _composite_kernel(s_ref, alpha_hbm, color_hbm, canvas_ref, den_ref,
                      abuf, cbuf, T_ref, C_ref, D_ref, cnt_ref, sem,
                      *, nb, n, w):
    fast_base = n - _CH0

    def start_fast(b):
        sl = b % 3
        pltpu.make_async_copy(alpha_hbm.at[b, pl.ds(fast_base, _CH0)],
                              abuf.at[sl], sem.at[sl, 0]).start()
        pltpu.make_async_copy(color_hbm.at[b, pl.ds(fast_base, _CH0)],
                              cbuf.at[sl], sem.at[sl, 1]).start()

    def wait_fast(b):
        sl = b % 3
        pltpu.make_async_copy(alpha_hbm.at[b, pl.ds(fast_base, _CH0)],
                              abuf.at[sl], sem.at[sl, 0]).wait()
        pltpu.make_async_copy(color_hbm.at[b, pl.ds(fast_base, _CH0)],
                              cbuf.at[sl], sem.at[sl, 1]).wait()

    start_fast(0)
    if nb > 1:
        start_fast(1)

    for b in range(nb):
        if b + 2 < nb:
            start_fast(b + 2)
        wait_fast(b)

        mincnt = _fast_chunk(s_ref, abuf, cbuf, T_ref, C_ref, D_ref,
                             cnt_ref, b, b % 3, fast_base, w)

        hbm_refs = (alpha_hbm, color_hbm)
        bufs = (abuf, cbuf)
        acc_refs = (T_ref, C_ref, D_ref, cnt_ref)

        @pl.when(mincnt < _K)
        def _continue_descending():
            _slow_pass(s_ref, hbm_refs, bufs, acc_refs, sem, b, b % 3,
                       n_start=0, n_count=fast_base, descending=True,
                       take_visible=True, with_density=True, w=w)

            @pl.when(jnp.min(cnt_ref[...]) < _K)
            def _tie_fill():
                _slow_pass(s_ref, hbm_refs, bufs, acc_refs, sem, b, b % 3,
                           n_start=0, n_count=n, descending=False,
                           take_visible=False, with_density=False, w=w)

        T = T_ref[...]
        canvas_ref[b, 0] = C_ref[0] + T
        canvas_ref[b, 1] = C_ref[1] + T
        canvas_ref[b, 2] = C_ref[2] + T
        den_ref[b, 0] = D_ref[...] + T


@jax.jit
def kernel(color_stroke, alpha, params):
    nb, n = color_stroke.shape[0], color_stroke.shape[1]
    w = color_stroke.shape[-1]

    kfn = functools.partial(_composite_kernel, nb=nb, n=n, w=w)

    canvas, den = pl.pallas_call(
        kfn,
        grid=(1,),
        in_specs=[
            pl.BlockSpec((nb, n, 8), lambda i: (0, 0, 0)),
            pl.BlockSpec(memory_space=pl.ANY),
            pl.BlockSpec(memory_space=pl.ANY),
        ],
        out_specs=[
            pl.BlockSpec((nb, 3, w, w), lambda i: (0, 0, 0, 0)),
            pl.BlockSpec((nb, 1, w, w), lambda i: (0, 0, 0, 0)),
        ],
        out_shape=[
            jax.ShapeDtypeStruct((nb, 3, w, w), jnp.float32),
            jax.ShapeDtypeStruct((nb, 1, w, w), jnp.float32),
        ],
        scratch_shapes=[
            pltpu.VMEM((3, _CH0, 1, w, w), jnp.float32),
            pltpu.VMEM((3, _CH0, 3, w, w), jnp.float32),
            pltpu.VMEM((w, w), jnp.float32),
            pltpu.VMEM((3, w, w), jnp.float32),
            pltpu.VMEM((w, w), jnp.float32),
            pltpu.VMEM((w, w), jnp.int32),
            pltpu.SemaphoreType.DMA((3, 2)),
        ],
        compiler_params=pltpu.CompilerParams(
            dimension_semantics=("arbitrary",),
        ),
    )(params, alpha, color_stroke)

    return (canvas, den)
